# baseline (speedup 1.0000x reference)
import jax
import jax.numpy as jnp
from jax.experimental import pallas as pl
from jax.experimental.pallas import tpu as pltpu

_T = 12
_CW = 128


def _fused_body(a_ref, q_ref, u_ref, cw_ref, cb_ref, o_ref, af_ref, dacc_ref):
    c = pl.program_id(0)
    nchunks = pl.num_programs(0)
    n = a_ref.shape[0]

    blk = a_ref[...]
    rown = jax.lax.broadcasted_iota(jnp.int32, (n, _CW), 0)
    coln = jax.lax.broadcasted_iota(jnp.int32, (n, _CW), 1) + c * _CW
    blk0 = jnp.where((coln != rown) & (coln < n), blk, 0.0)
    af_ref[c] = blk0

    psum = jnp.sum(blk0, axis=1, keepdims=True)

    @pl.when(c == 0)
    def _init():
        dacc_ref[...] = psum

    @pl.when(c > 0)
    def _acc():
        dacc_ref[...] = dacc_ref[...] + psum

    @pl.when(c == nchunks - 1)
    def _final():
        deg = dacc_ref[...]
        d = jnp.where(deg > 0, jax.lax.rsqrt(deg), 0.0)
        q = q_ref[...]
        u1 = u_ref[0].astype(jnp.float32)
        u2 = u_ref[1].astype(jnp.float32)
        npad = nchunks * _CW - n

        def smul(v):
            vs = jnp.concatenate(
                [d * v, jnp.zeros((npad, v.shape[1]), v.dtype)], axis=0)
            acc = jnp.zeros((n, v.shape[1]), jnp.float32)
            for j in range(nchunks):
                acc = acc + jnp.dot(af_ref[j], vs[j * _CW:(j + 1) * _CW],
                                    preferred_element_type=jnp.float32)
            return -d * acc

        w = smul(u1 + 2.0 * smul(u2))
        out = jnp.maximum(q + w, 0.0)

        z = jnp.zeros((n, 1), dtype=out.dtype)
        left = jnp.concatenate([z, out[:, :-1]], axis=1)
        right = jnp.concatenate([out[:, 1:], z], axis=1)
        colt = jax.lax.broadcasted_iota(jnp.int32, (1, out.shape[1]), 1) % _T
        mfirst = (colt != 0).astype(out.dtype)
        mlast = (colt != _T - 1).astype(out.dtype)
        cw = cw_ref[...]
        y = (cw[:, 1:2] * out
             + cw[:, 0:1] * (mfirst * left)
             + cw[:, 2:3] * (mlast * right)
             + cb_ref[0, 0])
        o_ref[...] = jnp.maximum(y, 0.0)


def kernel(X, A, W, b_gcn, conv_w, conv_b):
    B, N, _, T1 = X.shape
    K, _, T2 = W.shape
    BT = B * T2
    x3 = X.reshape(B, N, T1)
    Wq = jnp.stack([W[0] - W[2], W[1], W[2]])
    e = jnp.einsum('bnt,ktu->knbu', x3, Wq).reshape(K, N, BT)
    qf = e[0] + jnp.tile(b_gcn, B)[None, :]
    ub = e[1:].astype(jnp.bfloat16)

    nchunks = -(-N // _CW)
    y = pl.pallas_call(
        _fused_body,
        grid=(nchunks,),
        in_specs=[
            pl.BlockSpec((N, _CW), lambda c: (0, c)),
            pl.BlockSpec((N, BT), lambda c: (0, 0)),
            pl.BlockSpec((2, N, BT), lambda c: (0, 0, 0)),
            pl.BlockSpec((1, K), lambda c: (0, 0)),
            pl.BlockSpec((1, 1), lambda c: (0, 0)),
        ],
        out_specs=pl.BlockSpec((N, BT), lambda c: (0, 0)),
        out_shape=jax.ShapeDtypeStruct((N, BT), X.dtype),
        scratch_shapes=[
            pltpu.VMEM((nchunks, N, _CW), jnp.float32),
            pltpu.VMEM((N, 1), jnp.float32),
        ],
    )(A, qf, ub, conv_w.reshape(1, K), conv_b.reshape(1, 1))
    return y.reshape(N, B, T2).transpose(1, 0, 2).reshape(B, N, 1, T2)


# speedup vs baseline: 1.7648x; 1.7648x over previous
import jax
import jax.numpy as jnp
from jax.experimental import pallas as pl

_T = 12
_SEG = 256


def _fused_body(a_ref, x_ref, w_ref, bg_ref, cw_ref, cb_ref, o_ref):
    n = a_ref.shape[0]
    B = x_ref.shape[1] // _T
    BT = B * _T

    w = w_ref[...]
    rb = jax.lax.broadcasted_iota(jnp.int32, (BT, BT), 0) // _T
    cb = jax.lax.broadcasted_iota(jnp.int32, (BT, BT), 1) // _T
    blockmask = rb == cb
    zseg = jnp.zeros((BT, _SEG - BT), dtype=w.dtype)

    def bd(wk):
        tile = jnp.concatenate([jnp.concatenate([wk] * B, axis=1)] * B, axis=0)
        return jnp.concatenate(
            [jnp.where(blockmask, tile, 0.0), zseg], axis=1)

    WH = jnp.concatenate([bd(w[0] - w[2]), bd(w[1]), bd(w[2])], axis=1)

    x = x_ref[...]
    ual = jnp.dot(x, WH, preferred_element_type=jnp.float32)
    q = ual[:, 0:BT]
    u1 = ual[:, _SEG:_SEG + BT]
    u2 = ual[:, 2 * _SEG:2 * _SEG + BT]

    A = a_ref[...]
    rown = jax.lax.broadcasted_iota(jnp.int32, (n, n), 0)
    coln = jax.lax.broadcasted_iota(jnp.int32, (n, n), 1)
    A0 = jnp.where(rown == coln, 0.0, A)
    deg = jnp.sum(A0, axis=1, keepdims=True)
    d = jnp.where(deg > 0, jax.lax.rsqrt(deg), 0.0)

    v = -d * jnp.dot(A0, d * u2, preferred_element_type=jnp.float32)
    p = u1 + 2.0 * v
    w2 = -d * jnp.dot(A0, d * p, preferred_element_type=jnp.float32)
    bias = jnp.concatenate([bg_ref[...]] * B, axis=1)
    out = jnp.maximum(q + w2 + bias, 0.0)

    z = jnp.zeros((n, 1), dtype=out.dtype)
    left = jnp.concatenate([z, out[:, :-1]], axis=1)
    right = jnp.concatenate([out[:, 1:], z], axis=1)
    colt = jax.lax.broadcasted_iota(jnp.int32, (1, BT), 1) % _T
    mfirst = (colt != 0).astype(out.dtype)
    mlast = (colt != _T - 1).astype(out.dtype)
    cw = cw_ref[...]
    y = (cw[:, 1:2] * out
         + cw[:, 0:1] * (mfirst * left)
         + cw[:, 2:3] * (mlast * right)
         + cb_ref[0, 0])
    o_ref[...] = jnp.maximum(y, 0.0)


def kernel(X, A, W, b_gcn, conv_w, conv_b):
    B, N, _, T1 = X.shape
    K, _, T2 = W.shape
    xt = X.reshape(B, N, T1).transpose(1, 0, 2).reshape(N, B * T1)
    y = pl.pallas_call(
        _fused_body,
        out_shape=jax.ShapeDtypeStruct((N, B * T2), X.dtype),
    )(A, xt, W, b_gcn.reshape(1, T2), conv_w.reshape(1, K),
      conv_b.reshape(1, 1))
    return y.reshape(N, B, T2).transpose(1, 0, 2).reshape(B, N, 1, T2)
